# baseline (speedup 1.0000x reference)
import jax
import jax.numpy as jnp
from jax import lax
from jax.experimental import pallas as pl
from jax.experimental.pallas import tpu as pltpu
from jax.experimental.pallas import tpu_sc as plsc

N_NODES = 100000
D = 128
NUM_SEGMENTS = 1024
SEG_PAD = NUM_SEGMENTS + 16
EPS = 1e-5

NC = 2
NS = 16
NW = NC * NS
R = 160
NCHUNK = N_NODES // R
NG = R // 16
NV = D // 16
FMAX = 256
FG = FMAX // 16
CNTW = 16


def _sc_partial_sums(x3, b2, z_acc, z_cnt):
    mesh = plsc.VectorSubcoreMesh(core_axis_name="c", subcore_axis_name="s")

    @pl.kernel(
        out_type=[
            jax.ShapeDtypeStruct((NC, NUM_SEGMENTS, D), jnp.float32),
            jax.ShapeDtypeStruct((NC, NUM_SEGMENTS, CNTW), jnp.float32),
        ],
        mesh=mesh,
        scratch_types=[
            pltpu.VMEM((2, R, D), jnp.float32),
            pltpu.VMEM((2, R), jnp.int32),
            pltpu.VMEM((FMAX, D), jnp.float32),
            pltpu.VMEM((1, D), jnp.float32),
            pltpu.VMEM((FMAX, CNTW), jnp.float32),
            pltpu.VMEM((FG, 16), jnp.int32),
            pltpu.VMEM((FG, 16), jnp.float32),
            pltpu.VMEM_SHARED((SEG_PAD, D), jnp.float32),
            pltpu.VMEM_SHARED((SEG_PAD, CNTW), jnp.float32),
            pltpu.SemaphoreType.DMA((2,)),
            pltpu.SemaphoreType.DMA,
        ],
    )
    def k(x_hbm, b_hbm, z_hbm, zc_hbm, acc_out, cnt_out,
          xbuf, idsv, flushbuf, accbuf, cntbuf, fbidx, fbcnt,
          acc_sh, cnt_sh, sem_in, sem_sc):
        cid = lax.axis_index("c")
        sid = lax.axis_index("s")
        wid = sid * NC + cid
        i32 = jnp.int32
        f32 = jnp.float32
        lanes = lax.iota(i32, 16)
        zeros16 = jnp.zeros((16,), f32)

        @pl.when(sid == 0)
        def _():
            pltpu.sync_copy(z_hbm, acc_sh.at[pl.ds(0, NUM_SEGMENTS)])
            pltpu.sync_copy(z_hbm.at[pl.ds(0, 16)],
                            acc_sh.at[pl.ds(NUM_SEGMENTS, 16)])
            pltpu.sync_copy(zc_hbm, cnt_sh.at[pl.ds(0, NUM_SEGMENTS)])
            pltpu.sync_copy(zc_hbm.at[pl.ds(0, 16)],
                            cnt_sh.at[pl.ds(NUM_SEGMENTS, 16)])

        pltpu.sync_copy(z_hbm.at[pl.ds(0, FMAX)], flushbuf)
        pltpu.sync_copy(zc_hbm.at[pl.ds(0, FMAX)], cntbuf)
        for kk in range(NV):
            accbuf[0, pl.ds(kk * 16, 16)] = zeros16
        for t in range(FG):
            fbidx[t, pl.ds(0, 16)] = NUM_SEGMENTS + lanes
            fbcnt[t, pl.ds(0, 16)] = zeros16
        plsc.subcore_barrier()

        lo = (wid * NCHUNK) // NW
        hi = ((wid + 1) * NCHUNK) // NW

        def start_in(chunk, b):
            pltpu.async_copy(x_hbm.at[pl.ds(chunk, 1)], xbuf.at[pl.ds(b, 1)],
                             sem_in.at[b])
            pltpu.async_copy(b_hbm.at[pl.ds(chunk, 1)], idsv.at[pl.ds(b, 1)],
                             sem_in.at[b])

        def wait_in(chunk, b):
            pltpu.make_async_copy(x_hbm.at[pl.ds(chunk, 1)],
                                  xbuf.at[pl.ds(b, 1)], sem_in.at[b]).wait()
            pltpu.make_async_copy(b_hbm.at[pl.ds(chunk, 1)],
                                  idsv.at[pl.ds(b, 1)], sem_in.at[b]).wait()

        def flush_acc(fc, run_id, run_len):
            for kk in range(NV):
                flushbuf[fc, pl.ds(kk * 16, 16)] = accbuf[0, pl.ds(kk * 16, 16)]
                accbuf[0, pl.ds(kk * 16, 16)] = zeros16
            cntbuf[fc, pl.ds(0, 16)] = jnp.where(lanes == 0, run_len, 0.0)
            fg = fc // 16
            fl = fc % 16
            sel = lanes == fl
            fbidx[fg, pl.ds(0, 16)] = jnp.where(sel, run_id,
                                                fbidx[fg, pl.ds(0, 16)])
            fbcnt[fg, pl.ds(0, 16)] = jnp.where(sel, run_len,
                                                fbcnt[fg, pl.ds(0, 16)])
            return fc + 1

        def scatter_out(fc):
            ngrp = (fc + 15) // 16
            for t in range(FG):
                @pl.when(t < ngrp)
                def _():
                    pltpu.async_copy(flushbuf.at[pl.ds(t * 16, 16)],
                                     acc_sh.at[fbidx.at[t]], sem_sc,
                                     add=True).wait()
                    pltpu.async_copy(cntbuf.at[pl.ds(t * 16, 16)],
                                     cnt_sh.at[fbidx.at[t]], sem_sc,
                                     add=True).wait()

        def reset_flush():
            pltpu.sync_copy(z_hbm.at[pl.ds(0, FMAX)], flushbuf)
            pltpu.sync_copy(zc_hbm.at[pl.ds(0, FMAX)], cntbuf)
            for t in range(FG):
                fbidx[t, pl.ds(0, 16)] = NUM_SEGMENTS + lanes
                fbcnt[t, pl.ds(0, 16)] = zeros16

        start_in(lo, 0)
        start_in(lo + 1, 1)

        def chunk_body(chunk, carry):
            b = (chunk - lo) % 2
            wait_in(chunk, b)

            def group_body(g, carry2):
                run_id, run_len, fc = carry2
                idv = idsv[b, pl.ds(g * 16, 16)]
                base = g * 16

                def fast():
                    for kk in range(NV):
                        ds = pl.ds(kk * 16, 16)
                        gs = xbuf[b, base, ds]
                        for r in range(1, 16):
                            gs = gs + xbuf[b, base + r, ds]
                        accbuf[0, ds] = accbuf[0, ds] + gs
                    return run_id, run_len + 16.0, fc

                uniform = jnp.logical_and(idv[0] == run_id,
                                          idv[15] == run_id)

                def slow():
                    rid, rlen, f = run_id, run_len, fc
                    for r in range(16):
                        idr = idv[r]
                        is_new = idr != rid

                        def fl(f=f, rid=rid, rlen=rlen):
                            return flush_acc(f, rid, rlen)

                        f = lax.cond(jnp.logical_and(is_new, rlen > 0.0),
                                     fl, lambda f=f: f)
                        for kk in range(NV):
                            ds = pl.ds(kk * 16, 16)
                            accbuf[0, ds] = (accbuf[0, ds]
                                             + xbuf[b, base + r, ds])
                        rlen = jnp.where(is_new, 1.0, rlen + 1.0)
                        rid = idr
                    return rid, rlen, f

                return lax.cond(uniform, fast, slow)

            carry = lax.fori_loop(0, NG, group_body, carry)

            @pl.when(chunk + 2 < hi)
            def _():
                start_in(chunk + 2, b)

            run_id, run_len, fc = carry

            def mid_flush():
                scatter_out(fc)
                reset_flush()
                return 0

            fc = lax.cond(fc >= FMAX - R, mid_flush, lambda: fc)
            return (run_id, run_len, fc)

        init = (jnp.int32(-1), jnp.float32(0.0), jnp.int32(0))
        run_id, run_len, fc = lax.fori_loop(lo, hi, chunk_body, init)

        fc = lax.cond(run_len > 0.0,
                      lambda: flush_acc(fc, run_id, run_len),
                      lambda: fc)
        scatter_out(fc)

        plsc.subcore_barrier()

        @pl.when(sid == 0)
        def _():
            pltpu.sync_copy(acc_sh.at[pl.ds(0, NUM_SEGMENTS)],
                            acc_out.at[cid])
            pltpu.sync_copy(cnt_sh.at[pl.ds(0, NUM_SEGMENTS)],
                            cnt_out.at[cid])

    return k(x3, b2, z_acc, z_cnt)


def _finish(acc_ref, cnt_ref, w_ref, b_ref, o_ref):
    s = acc_ref[0] + acc_ref[1]
    c = cnt_ref[0, :, 0:1] + cnt_ref[1, :, 0:1]
    h = s / jnp.maximum(c, 1.0)
    mu = jnp.mean(h, axis=1, keepdims=True)
    var = jnp.mean((h - mu) ** 2, axis=1, keepdims=True)
    o_ref[...] = (h - mu) * lax.rsqrt(var + EPS) * w_ref[0] + b_ref[0]


def kernel(x, batch, ln_weight, ln_bias):
    z_acc = jnp.zeros((NUM_SEGMENTS, D), dtype=jnp.float32)
    z_cnt = jnp.zeros((NUM_SEGMENTS, CNTW), dtype=jnp.float32)
    acc_p, cnt_p = _sc_partial_sums(
        x.reshape(NCHUNK, R, D),
        batch.astype(jnp.int32).reshape(NCHUNK, R), z_acc, z_cnt)

    return pl.pallas_call(
        _finish,
        out_shape=jax.ShapeDtypeStruct((NUM_SEGMENTS, D), jnp.float32),
    )(acc_p, cnt_p, ln_weight.reshape(1, D), ln_bias.reshape(1, D))


# speedup vs baseline: 1.0594x; 1.0594x over previous
import jax
import jax.numpy as jnp
from jax import lax
from jax.experimental import pallas as pl
from jax.experimental.pallas import tpu as pltpu
from jax.experimental.pallas import tpu_sc as plsc

N_NODES = 100000
D = 128
NUM_SEGMENTS = 1024
SEG_PAD = NUM_SEGMENTS + 16
EPS = 1e-5

NC = 2
NS = 16
NW = NC * NS
R = 160
NCHUNK = N_NODES // R
NG = R // 16
NV = D // 16
FMAX = 256
FG = FMAX // 16
CNTW = 16


def _sc_partial_sums(x3, b2, z_acc, z_cnt):
    mesh = plsc.VectorSubcoreMesh(core_axis_name="c", subcore_axis_name="s")

    @pl.kernel(
        out_type=[
            jax.ShapeDtypeStruct((NC, NUM_SEGMENTS, D), jnp.float32),
            jax.ShapeDtypeStruct((NC, NUM_SEGMENTS, CNTW), jnp.float32),
        ],
        mesh=mesh,
        scratch_types=[
            pltpu.VMEM((2, R, D), jnp.float32),
            pltpu.VMEM((2, R), jnp.int32),
            pltpu.VMEM((FMAX, D), jnp.float32),
            pltpu.VMEM((1, D), jnp.float32),
            pltpu.VMEM((FMAX, CNTW), jnp.float32),
            pltpu.VMEM((FG, 16), jnp.int32),
            pltpu.VMEM((FG, 16), jnp.float32),
            pltpu.VMEM_SHARED((SEG_PAD, D), jnp.float32),
            pltpu.VMEM_SHARED((SEG_PAD, CNTW), jnp.float32),
            pltpu.SemaphoreType.DMA((2,)),
            pltpu.SemaphoreType.DMA,
        ],
    )
    def k(x_hbm, b_hbm, z_hbm, zc_hbm, acc_out, cnt_out,
          xbuf, idsv, flushbuf, accbuf, cntbuf, fbidx, fbcnt,
          acc_sh, cnt_sh, sem_in, sem_sc):
        cid = lax.axis_index("c")
        sid = lax.axis_index("s")
        wid = sid * NC + cid
        i32 = jnp.int32
        f32 = jnp.float32
        lanes = lax.iota(i32, 16)
        zeros16 = jnp.zeros((16,), f32)

        @pl.when(sid == 0)
        def _():
            pltpu.sync_copy(z_hbm, acc_sh.at[pl.ds(0, NUM_SEGMENTS)])
            pltpu.sync_copy(z_hbm.at[pl.ds(0, 16)],
                            acc_sh.at[pl.ds(NUM_SEGMENTS, 16)])
            pltpu.sync_copy(zc_hbm, cnt_sh.at[pl.ds(0, NUM_SEGMENTS)])
            pltpu.sync_copy(zc_hbm.at[pl.ds(0, 16)],
                            cnt_sh.at[pl.ds(NUM_SEGMENTS, 16)])

        pltpu.sync_copy(z_hbm.at[pl.ds(0, FMAX)], flushbuf)
        pltpu.sync_copy(zc_hbm.at[pl.ds(0, FMAX)], cntbuf)
        for kk in range(NV):
            accbuf[0, pl.ds(kk * 16, 16)] = zeros16
        for t in range(FG):
            fbidx[t, pl.ds(0, 16)] = NUM_SEGMENTS + lanes
            fbcnt[t, pl.ds(0, 16)] = zeros16
        plsc.subcore_barrier()

        lo = (wid * NCHUNK) // NW
        hi = ((wid + 1) * NCHUNK) // NW

        def start_in(chunk, b):
            pltpu.async_copy(x_hbm.at[pl.ds(chunk, 1)], xbuf.at[pl.ds(b, 1)],
                             sem_in.at[b])
            pltpu.async_copy(b_hbm.at[pl.ds(chunk, 1)], idsv.at[pl.ds(b, 1)],
                             sem_in.at[b])

        def wait_in(chunk, b):
            pltpu.make_async_copy(x_hbm.at[pl.ds(chunk, 1)],
                                  xbuf.at[pl.ds(b, 1)], sem_in.at[b]).wait()
            pltpu.make_async_copy(b_hbm.at[pl.ds(chunk, 1)],
                                  idsv.at[pl.ds(b, 1)], sem_in.at[b]).wait()

        def flush_acc(fc, run_id, run_len):
            for kk in range(NV):
                flushbuf[fc, pl.ds(kk * 16, 16)] = accbuf[0, pl.ds(kk * 16, 16)]
                accbuf[0, pl.ds(kk * 16, 16)] = zeros16
            cntbuf[fc, pl.ds(0, 16)] = jnp.where(lanes == 0, run_len, 0.0)
            fg = fc // 16
            fl = fc % 16
            sel = lanes == fl
            fbidx[fg, pl.ds(0, 16)] = jnp.where(sel, run_id,
                                                fbidx[fg, pl.ds(0, 16)])
            fbcnt[fg, pl.ds(0, 16)] = jnp.where(sel, run_len,
                                                fbcnt[fg, pl.ds(0, 16)])
            return fc + 1

        def scatter_out(fc):
            ngrp = (fc + 15) // 16
            for t in range(FG):
                @pl.when(t < ngrp)
                def _():
                    pltpu.async_copy(flushbuf.at[pl.ds(t * 16, 16)],
                                     acc_sh.at[fbidx.at[t]], sem_sc,
                                     add=True).wait()
                    pltpu.async_copy(cntbuf.at[pl.ds(t * 16, 16)],
                                     cnt_sh.at[fbidx.at[t]], sem_sc,
                                     add=True).wait()

        def reset_flush():
            pltpu.sync_copy(z_hbm.at[pl.ds(0, FMAX)], flushbuf)
            pltpu.sync_copy(zc_hbm.at[pl.ds(0, FMAX)], cntbuf)
            for t in range(FG):
                fbidx[t, pl.ds(0, 16)] = NUM_SEGMENTS + lanes
                fbcnt[t, pl.ds(0, 16)] = zeros16

        start_in(lo, 0)
        start_in(lo + 1, 1)

        def chunk_body(chunk, carry):
            b = (chunk - lo) % 2
            wait_in(chunk, b)

            def group_body(g, carry2):
                run_id, run_len, fc = carry2
                idv = idsv[b, pl.ds(g * 16, 16)]
                base = g * 16

                def fast():
                    for kk in range(NV):
                        ds = pl.ds(kk * 16, 16)
                        vs = [xbuf[b, base + r, ds] for r in range(16)]
                        while len(vs) > 1:
                            vs = [vs[i] + vs[i + 1]
                                  for i in range(0, len(vs), 2)]
                        accbuf[0, ds] = accbuf[0, ds] + vs[0]
                    return run_id, run_len + 16.0, fc

                uniform = jnp.logical_and(idv[0] == run_id,
                                          idv[15] == run_id)

                def slow():
                    rid, rlen, f = run_id, run_len, fc
                    for r in range(16):
                        idr = idv[r]
                        is_new = idr != rid

                        def fl(f=f, rid=rid, rlen=rlen):
                            return flush_acc(f, rid, rlen)

                        f = lax.cond(jnp.logical_and(is_new, rlen > 0.0),
                                     fl, lambda f=f: f)
                        for kk in range(NV):
                            ds = pl.ds(kk * 16, 16)
                            accbuf[0, ds] = (accbuf[0, ds]
                                             + xbuf[b, base + r, ds])
                        rlen = jnp.where(is_new, 1.0, rlen + 1.0)
                        rid = idr
                    return rid, rlen, f

                return lax.cond(uniform, fast, slow)

            carry = lax.fori_loop(0, NG, group_body, carry)

            @pl.when(chunk + 2 < hi)
            def _():
                start_in(chunk + 2, b)

            run_id, run_len, fc = carry

            def mid_flush():
                scatter_out(fc)
                reset_flush()
                return 0

            fc = lax.cond(fc >= FMAX - R, mid_flush, lambda: fc)
            return (run_id, run_len, fc)

        init = (jnp.int32(-1), jnp.float32(0.0), jnp.int32(0))
        run_id, run_len, fc = lax.fori_loop(lo, hi, chunk_body, init)

        fc = lax.cond(run_len > 0.0,
                      lambda: flush_acc(fc, run_id, run_len),
                      lambda: fc)
        scatter_out(fc)

        plsc.subcore_barrier()

        @pl.when(sid == 0)
        def _():
            pltpu.sync_copy(acc_sh.at[pl.ds(0, NUM_SEGMENTS)],
                            acc_out.at[cid])
            pltpu.sync_copy(cnt_sh.at[pl.ds(0, NUM_SEGMENTS)],
                            cnt_out.at[cid])

    return k(x3, b2, z_acc, z_cnt)


def _finish(acc_ref, cnt_ref, w_ref, b_ref, o_ref):
    s = acc_ref[0] + acc_ref[1]
    c = cnt_ref[0, :, 0:1] + cnt_ref[1, :, 0:1]
    h = s / jnp.maximum(c, 1.0)
    mu = jnp.mean(h, axis=1, keepdims=True)
    var = jnp.mean((h - mu) ** 2, axis=1, keepdims=True)
    o_ref[...] = (h - mu) * lax.rsqrt(var + EPS) * w_ref[0] + b_ref[0]


def kernel(x, batch, ln_weight, ln_bias):
    z_acc = jnp.zeros((NUM_SEGMENTS, D), dtype=jnp.float32)
    z_cnt = jnp.zeros((NUM_SEGMENTS, CNTW), dtype=jnp.float32)
    acc_p, cnt_p = _sc_partial_sums(
        x.reshape(NCHUNK, R, D),
        batch.astype(jnp.int32).reshape(NCHUNK, R), z_acc, z_cnt)

    return pl.pallas_call(
        _finish,
        out_shape=jax.ShapeDtypeStruct((NUM_SEGMENTS, D), jnp.float32),
    )(acc_p, cnt_p, ln_weight.reshape(1, D), ln_bias.reshape(1, D))


# speedup vs baseline: 1.7028x; 1.6073x over previous
import jax
import jax.numpy as jnp
from jax import lax
from jax.experimental import pallas as pl
from jax.experimental.pallas import tpu as pltpu
from jax.experimental.pallas import tpu_sc as plsc

N_NODES = 100000
D = 128
NUM_SEGMENTS = 1024
EPS = 1e-5

NC = 2
NS = 16
NW = NC * NS
R = 400
NCHUNK = N_NODES // R
SCW = 80
NSC = R // SCW
CNTW = 16


def _sc_partial_sums(x, batch3, ones_hbm, z_sums, z_cnt):
    mesh = plsc.VectorSubcoreMesh(core_axis_name="c", subcore_axis_name="s")

    @pl.kernel(
        out_type=[
            jax.ShapeDtypeStruct((NC, NUM_SEGMENTS, D), jnp.float32),
            jax.ShapeDtypeStruct((NC, NUM_SEGMENTS, CNTW), jnp.float32),
        ],
        mesh=mesh,
        scratch_types=[
            pltpu.VMEM((2, R, D), jnp.float32),
            pltpu.VMEM((2, NSC, SCW), jnp.int32),
            pltpu.VMEM((SCW, CNTW), jnp.float32),
            pltpu.VMEM_SHARED((NUM_SEGMENTS, D), jnp.float32),
            pltpu.VMEM_SHARED((NUM_SEGMENTS, CNTW), jnp.float32),
            pltpu.SemaphoreType.DMA,
            pltpu.SemaphoreType.DMA,
            pltpu.SemaphoreType.DMA,
        ],
    )
    def k(x_hbm, b_hbm, ones_h, zs_h, zc_h, sums_out, cnts_out,
          xbuf, idxbuf, onesbuf, sums_sh, cnts_sh, sem0, sem1, sem_sc):
        cid = lax.axis_index("c")
        sid = lax.axis_index("s")
        wid = sid * NC + cid
        sems = (sem0, sem1)

        @pl.when(sid == 0)
        def _():
            pltpu.sync_copy(zs_h, sums_sh)
            pltpu.sync_copy(zc_h, cnts_sh)

        pltpu.sync_copy(ones_h, onesbuf)
        plsc.subcore_barrier()

        lo = (wid * NCHUNK) // NW
        hi = ((wid + 1) * NCHUNK) // NW

        def start_in(chunk, b):
            pltpu.async_copy(x_hbm.at[pl.ds(chunk * R, R)], xbuf.at[b], sems[b])
            pltpu.async_copy(b_hbm.at[chunk], idxbuf.at[b], sems[b])

        def wait_in(chunk, b):
            pltpu.make_async_copy(x_hbm.at[pl.ds(chunk * R, R)], xbuf.at[b],
                                  sems[b]).wait()
            pltpu.make_async_copy(b_hbm.at[chunk], idxbuf.at[b],
                                  sems[b]).wait()

        start_in(lo, 0)
        start_in(lo + 1, 1)

        n_outer = (hi - lo + 1) // 2

        def body(kk, carry):
            i = lo + 2 * kk
            for b in range(2):
                chunk = i + b

                @pl.when(chunk < hi)
                def _():
                    wait_in(chunk, b)
                    xv = xbuf.at[b]
                    iv = idxbuf.at[b]
                    descs = []
                    for j in range(NSC):
                        descs.append(pltpu.async_copy(
                            xv.at[pl.ds(j * SCW, SCW)],
                            sums_sh.at[iv.at[j]], sem_sc, add=True))
                        descs.append(pltpu.async_copy(
                            onesbuf, cnts_sh.at[iv.at[j]], sem_sc, add=True))
                    for d in descs:
                        d.wait()

                    @pl.when(chunk + 2 < hi)
                    def _():
                        start_in(chunk + 2, b)
            return carry

        lax.fori_loop(0, n_outer, body, 0)
        plsc.subcore_barrier()

        @pl.when(sid == 0)
        def _():
            pltpu.sync_copy(sums_sh, sums_out.at[cid])
            pltpu.sync_copy(cnts_sh, cnts_out.at[cid])

    return k(x, batch3, ones_hbm, z_sums, z_cnt)


def _finish(sums_ref, cnts_ref, w_ref, b_ref, o_ref):
    s = sums_ref[0] + sums_ref[1]
    c = cnts_ref[0, :, 0:1] + cnts_ref[1, :, 0:1]
    h = s / jnp.maximum(c, 1.0)
    mu = jnp.mean(h, axis=1, keepdims=True)
    var = jnp.mean((h - mu) ** 2, axis=1, keepdims=True)
    o_ref[...] = (h - mu) * lax.rsqrt(var + EPS) * w_ref[0] + b_ref[0]


def kernel(x, batch, ln_weight, ln_bias):
    batch3 = batch.astype(jnp.int32).reshape(NCHUNK, NSC, SCW)
    ones_hbm = jnp.ones((SCW, CNTW), dtype=jnp.float32)
    z_sums = jnp.zeros((NUM_SEGMENTS, D), dtype=jnp.float32)
    z_cnt = jnp.zeros((NUM_SEGMENTS, CNTW), dtype=jnp.float32)

    sums_p, cnts_p = _sc_partial_sums(x, batch3, ones_hbm, z_sums, z_cnt)

    return pl.pallas_call(
        _finish,
        out_shape=jax.ShapeDtypeStruct((NUM_SEGMENTS, D), jnp.float32),
    )(sums_p, cnts_p, ln_weight.reshape(1, D), ln_bias.reshape(1, D))
